# baseline (speedup 1.0000x reference)
import functools

import jax
import jax.numpy as jnp
from jax import lax
from jax.experimental import pallas as pl
from jax.experimental.pallas import tpu as pltpu
from jax.experimental.pallas import tpu_sc as plsc

_B, _L, _D = 4096, 50, 32
_V = 1_000_000
_NC, _NS = 2, 16
_NW = _NC * _NS
_BPW = _B // _NW
_IPW = _BPW * _L
_QOFF = (0, 13, 26, 38)
_QLEN = (13, 13, 12, 12)
_NSL = 8
_SCALE = 1.0 / _L

_CW = 128
_NCHUNK = _V // _CW
_TAIL = _V - _NCHUNK * _CW
_RN = 4

_mesh = plsc.VectorSubcoreMesh(
    core_axis_name="c", subcore_axis_name="s", num_cores=_NC, num_subcores=_NS
)


@functools.partial(
    pl.kernel,
    out_type=jax.ShapeDtypeStruct((_V // 4, 128), jnp.float32),
    mesh=_mesh,
    scratch_types=[
        pltpu.VMEM((_RN, _D, _CW), jnp.float32),
        pltpu.VMEM((_RN, _CW // 4, 128), jnp.float32),
        pltpu.SemaphoreType.DMA((_RN,)),
        pltpu.SemaphoreType.DMA((_RN,)),
    ],
    compiler_params=pltpu.CompilerParams(needs_layout_passes=False),
)
def _repack(tab_hbm, tail_hbm, lin_hbm, in_v, pk_v, s_in, s_out):
    wid = lax.axis_index("s") * _NC + lax.axis_index("c")
    trip = _NCHUNK // _NW + jnp.where(wid < _NCHUNK % _NW, 1, 0)
    iota = lax.iota(jnp.int32, 16)

    def _cidx(i):
        return wid + _NW * i

    def _in_copy(i, sl):
        off = pl.multiple_of(_CW * _cidx(i), 128)
        return pltpu.make_async_copy(
            tab_hbm.at[:, pl.ds(off, _CW)], in_v.at[sl], s_in.at[sl]
        )

    def _out_copy(i, sl):
        off = pl.multiple_of((_CW // 4) * _cidx(i), 8)
        return pltpu.make_async_copy(
            pk_v.at[sl], lin_hbm.at[pl.ds(off, _CW // 4)], s_out.at[sl]
        )

    def _transpose(sl, width):
        slab = in_v.at[sl]
        for j in range(width):
            jv = jnp.full((16,), j, jnp.int32)
            g0 = plsc.load_gather(slab, [iota, jv])
            g1 = plsc.load_gather(slab, [iota + 16, jv])
            pk_v[sl, j // 4, pl.ds((j % 4) * 32, 16)] = g0
            pk_v[sl, j // 4, pl.ds((j % 4) * 32 + 16, 16)] = g1

    def _step(i):
        sl = lax.rem(i, _RN)
        _in_copy(i, sl).wait()

        @pl.when(i >= _RN)
        def _():
            _out_copy(i - _RN, sl).wait()

        _transpose(sl, _CW)
        _out_copy(i, sl).start()

        @pl.when(i + _RN < trip)
        def _():
            _in_copy(i + _RN, sl).start()

    for sl in range(_RN):
        _in_copy(sl, sl).start()

    @pl.loop(0, trip)
    def _main(i):
        _step(i)

    for sl in range(_RN):
        _out_copy(0, sl).wait()

    @pl.when(wid == _NW - 1)
    def _():
        pltpu.sync_copy(
            tail_hbm, lin_hbm.at[pl.ds(_NCHUNK * (_CW // 4), _TAIL // 4)]
        )


@functools.partial(
    pl.kernel,
    out_type=jax.ShapeDtypeStruct((_B * _D,), jnp.float32),
    mesh=_mesh,
    scratch_types=[
        pltpu.VMEM((_IPW + 16,), jnp.int32),
        pltpu.VMEM((_BPW * _D,), jnp.float32),
        pltpu.VMEM((_NSL, 16), jnp.int32),
        pltpu.VMEM((_NSL, 16, 128), jnp.float32),
        pltpu.SemaphoreType.DMA((_NSL,)),
    ],
    compiler_params=pltpu.CompilerParams(needs_layout_passes=False),
)
def _lookup_mean(idx_hbm, table_hbm, out_hbm, idx_v, out_v, lists_v, ring_v, sems):
    wid = lax.axis_index("s") * _NC + lax.axis_index("c")
    pltpu.sync_copy(idx_hbm.at[pl.ds(wid * _IPW, _IPW)], idx_v.at[pl.ds(0, _IPW)])
    iota = lax.iota(jnp.int32, 16)

    def _chunk(b, c):
        off = b * _L + _QOFF[c]
        return plsc.load_gather(idx_v, [jnp.full((16,), off, jnp.int32) + iota])

    def _issue(b, c, s):
        v = _chunk(b, c)
        lists_v[s, :] = v >> 2
        pltpu.async_copy(
            table_hbm.at[lists_v.at[s, pl.ds(0, _QLEN[c])]],
            ring_v.at[s, pl.ds(0, _QLEN[c])],
            sems.at[s],
        )

    def _consume(b, c, s, a0, a1):
        pltpu.make_async_copy(
            table_hbm.at[lists_v.at[s, pl.ds(0, _QLEN[c])]],
            ring_v.at[s, pl.ds(0, _QLEN[c])],
            sems.at[s],
        ).wait()
        v = _chunk(b, c)
        sub = v & 3
        for i in range(_QLEN[c]):
            si = sub[i] * 32
            a0 = a0 + ring_v[s, i, pl.ds(si, 16)]
            a1 = a1 + ring_v[s, i, pl.ds(si + 16, 16)]
        return a0, a1

    for p in range(2):
        for c in range(4):
            _issue(p, c, 4 * p + c)

    @pl.loop(0, _BPW - 2, step=2)
    def _main(b):
        for p in range(2):
            a0 = jnp.zeros((16,), jnp.float32)
            a1 = jnp.zeros((16,), jnp.float32)
            for c in range(4):
                s = 4 * p + c
                a0, a1 = _consume(b + p, c, s, a0, a1)
                _issue(b + p + 2, c, s)
            out_v[pl.ds((b + p) * _D, 16)] = a0 * _SCALE
            out_v[pl.ds((b + p) * _D + 16, 16)] = a1 * _SCALE

    for p in range(2):
        b = _BPW - 2 + p
        a0 = jnp.zeros((16,), jnp.float32)
        a1 = jnp.zeros((16,), jnp.float32)
        for c in range(4):
            a0, a1 = _consume(b, c, 4 * p + c, a0, a1)
        out_v[pl.ds(b * _D, 16)] = a0 * _SCALE
        out_v[pl.ds(b * _D + 16, 16)] = a1 * _SCALE

    pltpu.sync_copy(out_v, out_hbm.at[pl.ds(wid * _BPW * _D, _BPW * _D)])


def kernel(idx, embedding):
    idx1d = idx.reshape(-1)
    tail = embedding[_NCHUNK * _CW :, :].reshape(_TAIL // 4, 128)
    lin = _repack(embedding.T, tail)
    out = _lookup_mean(idx1d, lin)
    return out.reshape(_B, 1, _D)


# speedup vs baseline: 1.0004x; 1.0004x over previous
import functools

import jax
import jax.numpy as jnp
from jax import lax
from jax.experimental import pallas as pl
from jax.experimental.pallas import tpu as pltpu
from jax.experimental.pallas import tpu_sc as plsc

_B, _L, _D = 4096, 50, 32
_V = 1_000_000
_NC, _NS = 2, 16
_NW = _NC * _NS
_BPW = _B // _NW
_IPW = _BPW * _L
_QOFF = (0, 13, 26, 38)
_QLEN = (13, 13, 12, 12)
_NSL = 8
_SCALE = 1.0 / _L

_CW = 128
_NCHUNK = _V // _CW
_TAIL = _V - _NCHUNK * _CW
_RN = 4

_mesh = plsc.VectorSubcoreMesh(
    core_axis_name="c", subcore_axis_name="s", num_cores=_NC, num_subcores=_NS
)


@functools.partial(
    pl.kernel,
    out_type=jax.ShapeDtypeStruct((_V // 4, 128), jnp.float32),
    mesh=_mesh,
    scratch_types=[
        pltpu.VMEM((_RN, _D, _CW), jnp.float32),
        pltpu.VMEM((_RN, _CW // 4, 128), jnp.float32),
        pltpu.SemaphoreType.DMA((_RN,)),
        pltpu.SemaphoreType.DMA((_RN,)),
    ],
    compiler_params=pltpu.CompilerParams(needs_layout_passes=False),
)
def _repack(tab_hbm, tail_hbm, lin_hbm, in_v, pk_v, s_in, s_out):
    wid = lax.axis_index("s") * _NC + lax.axis_index("c")
    trip = _NCHUNK // _NW + jnp.where(wid < _NCHUNK % _NW, 1, 0)
    iota = lax.iota(jnp.int32, 16)

    def _cidx(i):
        return wid + _NW * i

    def _in_copy(i, sl):
        off = pl.multiple_of(_CW * _cidx(i), 128)
        return pltpu.make_async_copy(
            tab_hbm.at[:, pl.ds(off, _CW)], in_v.at[sl], s_in.at[sl]
        )

    def _out_copy(i, sl):
        off = pl.multiple_of((_CW // 4) * _cidx(i), 8)
        return pltpu.make_async_copy(
            pk_v.at[sl], lin_hbm.at[pl.ds(off, _CW // 4)], s_out.at[sl]
        )

    iota16 = iota + 16

    def _transpose(sl, width):
        slab = in_v.at[sl]
        for j0 in range(0, width, 8):
            gs = []
            for j in range(j0, j0 + 8):
                jv = jnp.full((16,), j, jnp.int32)
                gs.append(plsc.load_gather(slab, [iota, jv]))
                gs.append(plsc.load_gather(slab, [iota16, jv]))
            for k, j in enumerate(range(j0, j0 + 8)):
                pk_v[sl, j // 4, pl.ds((j % 4) * 32, 16)] = gs[2 * k]
                pk_v[sl, j // 4, pl.ds((j % 4) * 32 + 16, 16)] = gs[2 * k + 1]

    def _step(i, sl, first):
        _in_copy(i, sl).wait()

        @pl.when(jnp.logical_not(first))
        def _():
            _out_copy(i - _RN, sl).wait()

        _transpose(sl, _CW)
        _out_copy(i, sl).start()

        @pl.when(i + _RN < trip)
        def _():
            _in_copy(i + _RN, sl).start()

    for sl in range(_RN):
        _in_copy(sl, sl).start()

    @pl.loop(0, _NCHUNK // _NW // _RN)
    def _main(ii):
        for sl in range(_RN):
            _step(_RN * ii + sl, sl, ii == 0)

    @pl.when(wid < _NCHUNK % _NW)
    def _():
        i = _NCHUNK // _NW
        _in_copy(i, 0).wait()
        _out_copy(i - _RN, 0).wait()
        _transpose(0, _CW)
        _out_copy(i, 0).start()

    for sl in range(_RN):
        _out_copy(0, sl).wait()

    @pl.when(wid == _NW - 1)
    def _():
        pltpu.sync_copy(
            tail_hbm, lin_hbm.at[pl.ds(_NCHUNK * (_CW // 4), _TAIL // 4)]
        )


@functools.partial(
    pl.kernel,
    out_type=jax.ShapeDtypeStruct((_B * _D,), jnp.float32),
    mesh=_mesh,
    scratch_types=[
        pltpu.VMEM((_IPW + 16,), jnp.int32),
        pltpu.VMEM((_BPW * _D,), jnp.float32),
        pltpu.VMEM((_NSL, 16), jnp.int32),
        pltpu.VMEM((_NSL, 16, 128), jnp.float32),
        pltpu.SemaphoreType.DMA((_NSL,)),
    ],
    compiler_params=pltpu.CompilerParams(needs_layout_passes=False),
)
def _lookup_mean(idx_hbm, table_hbm, out_hbm, idx_v, out_v, lists_v, ring_v, sems):
    wid = lax.axis_index("s") * _NC + lax.axis_index("c")
    pltpu.sync_copy(idx_hbm.at[pl.ds(wid * _IPW, _IPW)], idx_v.at[pl.ds(0, _IPW)])
    iota = lax.iota(jnp.int32, 16)

    def _chunk(b, c):
        off = b * _L + _QOFF[c]
        return plsc.load_gather(idx_v, [jnp.full((16,), off, jnp.int32) + iota])

    def _issue(b, c, s):
        v = _chunk(b, c)
        lists_v[s, :] = v >> 2
        pltpu.async_copy(
            table_hbm.at[lists_v.at[s, pl.ds(0, _QLEN[c])]],
            ring_v.at[s, pl.ds(0, _QLEN[c])],
            sems.at[s],
        )

    def _consume(b, c, s, a0, a1):
        pltpu.make_async_copy(
            table_hbm.at[lists_v.at[s, pl.ds(0, _QLEN[c])]],
            ring_v.at[s, pl.ds(0, _QLEN[c])],
            sems.at[s],
        ).wait()
        v = _chunk(b, c)
        sub = v & 3
        for i in range(_QLEN[c]):
            si = sub[i] * 32
            a0 = a0 + ring_v[s, i, pl.ds(si, 16)]
            a1 = a1 + ring_v[s, i, pl.ds(si + 16, 16)]
        return a0, a1

    for p in range(2):
        for c in range(4):
            _issue(p, c, 4 * p + c)

    @pl.loop(0, _BPW - 2, step=2)
    def _main(b):
        for p in range(2):
            a0 = jnp.zeros((16,), jnp.float32)
            a1 = jnp.zeros((16,), jnp.float32)
            for c in range(4):
                s = 4 * p + c
                a0, a1 = _consume(b + p, c, s, a0, a1)
                _issue(b + p + 2, c, s)
            out_v[pl.ds((b + p) * _D, 16)] = a0 * _SCALE
            out_v[pl.ds((b + p) * _D + 16, 16)] = a1 * _SCALE

    for p in range(2):
        b = _BPW - 2 + p
        a0 = jnp.zeros((16,), jnp.float32)
        a1 = jnp.zeros((16,), jnp.float32)
        for c in range(4):
            a0, a1 = _consume(b, c, 4 * p + c, a0, a1)
        out_v[pl.ds(b * _D, 16)] = a0 * _SCALE
        out_v[pl.ds(b * _D + 16, 16)] = a1 * _SCALE

    pltpu.sync_copy(out_v, out_hbm.at[pl.ds(wid * _BPW * _D, _BPW * _D)])


def kernel(idx, embedding):
    idx1d = idx.reshape(-1)
    tail = embedding[_NCHUNK * _CW :, :].reshape(_TAIL // 4, 128)
    lin = _repack(embedding.T, tail)
    out = _lookup_mean(idx1d, lin)
    return out.reshape(_B, 1, _D)


# speedup vs baseline: 1.0624x; 1.0620x over previous
import functools

import jax
import jax.numpy as jnp
from jax import lax
from jax.experimental import pallas as pl
from jax.experimental.pallas import tpu as pltpu
from jax.experimental.pallas import tpu_sc as plsc

_B, _L, _D = 4096, 50, 32
_V = 1_000_000
_NC, _NS = 2, 16
_NW = _NC * _NS
_BPW = _B // _NW
_IPW = _BPW * _L
_QOFF = (0, 13, 26, 38)
_QLEN = (13, 13, 12, 12)
_NSL = 8
_SCALE = 1.0 / _L

_CW = 128
_NCHUNK = _V // _CW
_TAIL = _V - _NCHUNK * _CW
_RN = 4

_mesh = plsc.VectorSubcoreMesh(
    core_axis_name="c", subcore_axis_name="s", num_cores=_NC, num_subcores=_NS
)


@functools.partial(
    pl.kernel,
    out_type=jax.ShapeDtypeStruct((_V // 4, 128), jnp.float32),
    mesh=_mesh,
    scratch_types=[
        pltpu.VMEM((_RN, _D, _CW), jnp.float32),
        pltpu.VMEM((_RN, _CW // 4, 128), jnp.float32),
        pltpu.SemaphoreType.DMA((_RN,)),
        pltpu.SemaphoreType.DMA((_RN,)),
    ],
    compiler_params=pltpu.CompilerParams(needs_layout_passes=False),
)
def _repack(tab_hbm, tail_hbm, lin_hbm, in_v, pk_v, s_in, s_out):
    wid = lax.axis_index("s") * _NC + lax.axis_index("c")
    trip = _NCHUNK // _NW + jnp.where(wid < _NCHUNK % _NW, 1, 0)
    iota = lax.iota(jnp.int32, 16)

    def _cidx(i):
        return wid + _NW * i

    def _in_copy(i, sl):
        off = pl.multiple_of(_CW * _cidx(i), 128)
        return pltpu.make_async_copy(
            tab_hbm.at[:, pl.ds(off, _CW)], in_v.at[sl], s_in.at[sl]
        )

    def _out_copy(i, sl):
        off = pl.multiple_of((_CW // 4) * _cidx(i), 8)
        return pltpu.make_async_copy(
            pk_v.at[sl], lin_hbm.at[pl.ds(off, _CW // 4)], s_out.at[sl]
        )

    perm_l = iota >> 2
    perm_w = (iota & 3) * 32

    def _transpose(sl, width):
        for c in range(_D):
            wvec = perm_w + c
            for j0 in range(0, width, 16):
                g = in_v[sl, c, pl.ds(j0, 16)]
                plsc.store_scatter(pk_v.at[sl], [perm_l + (j0 // 4), wvec], g)

    def _step(i, sl, first):
        _in_copy(i, sl).wait()

        @pl.when(jnp.logical_not(first))
        def _():
            _out_copy(i - _RN, sl).wait()

        _transpose(sl, _CW)
        _out_copy(i, sl).start()

        @pl.when(i + _RN < trip)
        def _():
            _in_copy(i + _RN, sl).start()

    for sl in range(_RN):
        _in_copy(sl, sl).start()

    @pl.loop(0, _NCHUNK // _NW // _RN)
    def _main(ii):
        for sl in range(_RN):
            _step(_RN * ii + sl, sl, ii == 0)

    @pl.when(wid < _NCHUNK % _NW)
    def _():
        i = _NCHUNK // _NW
        _in_copy(i, 0).wait()
        _out_copy(i - _RN, 0).wait()
        _transpose(0, _CW)
        _out_copy(i, 0).start()

    for sl in range(_RN):
        _out_copy(0, sl).wait()

    @pl.when(wid == _NW - 1)
    def _():
        pltpu.sync_copy(
            tail_hbm, lin_hbm.at[pl.ds(_NCHUNK * (_CW // 4), _TAIL // 4)]
        )


@functools.partial(
    pl.kernel,
    out_type=jax.ShapeDtypeStruct((_B * _D,), jnp.float32),
    mesh=_mesh,
    scratch_types=[
        pltpu.VMEM((_IPW + 16,), jnp.int32),
        pltpu.VMEM((_BPW * _D,), jnp.float32),
        pltpu.VMEM((_NSL, 16), jnp.int32),
        pltpu.VMEM((_NSL, 16, 128), jnp.float32),
        pltpu.SemaphoreType.DMA((_NSL,)),
    ],
    compiler_params=pltpu.CompilerParams(needs_layout_passes=False),
)
def _lookup_mean(idx_hbm, table_hbm, out_hbm, idx_v, out_v, lists_v, ring_v, sems):
    wid = lax.axis_index("s") * _NC + lax.axis_index("c")
    pltpu.sync_copy(idx_hbm.at[pl.ds(wid * _IPW, _IPW)], idx_v.at[pl.ds(0, _IPW)])
    iota = lax.iota(jnp.int32, 16)

    def _chunk(b, c):
        off = b * _L + _QOFF[c]
        return plsc.load_gather(idx_v, [jnp.full((16,), off, jnp.int32) + iota])

    def _issue(b, c, s):
        v = _chunk(b, c)
        lists_v[s, :] = v >> 2
        pltpu.async_copy(
            table_hbm.at[lists_v.at[s, pl.ds(0, _QLEN[c])]],
            ring_v.at[s, pl.ds(0, _QLEN[c])],
            sems.at[s],
        )

    def _consume(b, c, s, a0, a1):
        pltpu.make_async_copy(
            table_hbm.at[lists_v.at[s, pl.ds(0, _QLEN[c])]],
            ring_v.at[s, pl.ds(0, _QLEN[c])],
            sems.at[s],
        ).wait()
        v = _chunk(b, c)
        sub = v & 3
        for i in range(_QLEN[c]):
            si = sub[i] * 32
            a0 = a0 + ring_v[s, i, pl.ds(si, 16)]
            a1 = a1 + ring_v[s, i, pl.ds(si + 16, 16)]
        return a0, a1

    for p in range(2):
        for c in range(4):
            _issue(p, c, 4 * p + c)

    @pl.loop(0, _BPW - 2, step=2)
    def _main(b):
        for p in range(2):
            a0 = jnp.zeros((16,), jnp.float32)
            a1 = jnp.zeros((16,), jnp.float32)
            for c in range(4):
                s = 4 * p + c
                a0, a1 = _consume(b + p, c, s, a0, a1)
                _issue(b + p + 2, c, s)
            out_v[pl.ds((b + p) * _D, 16)] = a0 * _SCALE
            out_v[pl.ds((b + p) * _D + 16, 16)] = a1 * _SCALE

    for p in range(2):
        b = _BPW - 2 + p
        a0 = jnp.zeros((16,), jnp.float32)
        a1 = jnp.zeros((16,), jnp.float32)
        for c in range(4):
            a0, a1 = _consume(b, c, 4 * p + c, a0, a1)
        out_v[pl.ds(b * _D, 16)] = a0 * _SCALE
        out_v[pl.ds(b * _D + 16, 16)] = a1 * _SCALE

    pltpu.sync_copy(out_v, out_hbm.at[pl.ds(wid * _BPW * _D, _BPW * _D)])


def kernel(idx, embedding):
    idx1d = idx.reshape(-1)
    tail = embedding[_NCHUNK * _CW :, :].reshape(_TAIL // 4, 128)
    lin = _repack(embedding.T, tail)
    out = _lookup_mean(idx1d, lin)
    return out.reshape(_B, 1, _D)


# speedup vs baseline: 1.4006x; 1.3184x over previous
import functools

import jax
import jax.numpy as jnp
from jax import lax
from jax.experimental import pallas as pl
from jax.experimental.pallas import tpu as pltpu
from jax.experimental.pallas import tpu_sc as plsc

_B, _L, _D = 4096, 50, 32
_NC, _NS = 2, 16
_NW = _NC * _NS
_BPW = _B // _NW
_NBUF = 4
_SCALE = 1.0 / _L

_mesh = plsc.VectorSubcoreMesh(
    core_axis_name="c", subcore_axis_name="s", num_cores=_NC, num_subcores=_NS
)


@functools.partial(
    pl.kernel,
    out_type=jax.ShapeDtypeStruct((_B, _D), jnp.float32),
    mesh=_mesh,
    scratch_types=[
        pltpu.VMEM((_BPW, _L), jnp.int32),
        pltpu.VMEM((_NBUF, _L, _D), jnp.float32),
        pltpu.VMEM((_BPW, _D), jnp.float32),
        pltpu.SemaphoreType.DMA((_NBUF,)),
    ],
    compiler_params=pltpu.CompilerParams(use_tc_tiling_on_sc=False),
)
def _lookup_mean(idx_hbm, table_hbm, out_hbm, idx_v, ring_v, out_v, sems):
    wid = lax.axis_index("s") * _NC + lax.axis_index("c")
    base = wid * _BPW
    pltpu.sync_copy(idx_hbm.at[pl.ds(base, _BPW)], idx_v)

    def _start(b, s):
        pltpu.async_copy(table_hbm.at[idx_v.at[b]], ring_v.at[s], sems.at[s])

    def _wait(b, s):
        pltpu.make_async_copy(
            table_hbm.at[idx_v.at[b]], ring_v.at[s], sems.at[s]
        ).wait()

    def _acc_row(b, s):
        _wait(b, s)
        a0 = ring_v[s, 0, 0:16]
        a1 = ring_v[s, 0, 16:32]
        for t in range(1, _L):
            a0 = a0 + ring_v[s, t, 0:16]
            a1 = a1 + ring_v[s, t, 16:32]
        return a0, a1

    for s in range(_NBUF):
        _start(s, s)

    @pl.loop(0, _BPW - _NBUF, step=_NBUF)
    def _main(b0):
        for s in range(_NBUF):
            b = b0 + s
            a0, a1 = _acc_row(b, s)
            _start(b + _NBUF, s)
            out_v[b, 0:16] = a0 * _SCALE
            out_v[b, 16:32] = a1 * _SCALE

    for s in range(_NBUF):
        b = (_BPW - _NBUF) + s
        a0, a1 = _acc_row(b, s)
        out_v[b, 0:16] = a0 * _SCALE
        out_v[b, 16:32] = a1 * _SCALE

    pltpu.sync_copy(out_v, out_hbm.at[pl.ds(base, _BPW)])


def kernel(idx, embedding):
    out = _lookup_mean(idx, embedding)
    return out[:, None, :]
